# baseline (speedup 1.0000x reference)
import functools

import jax
import jax.numpy as jnp
from jax import lax
from jax.experimental import pallas as pl
from jax.experimental.pallas import tpu as pltpu
from jax.experimental.pallas import tpu_sc as plsc

NC = 2
NS = 16
L = 16
CH = 128
NBUF = 2


def _rsqrt(x):
    i = lax.bitcast_convert_type(x, jnp.int32)
    i = jnp.int32(0x5F3759DF) - lax.shift_right_logical(i, 1)
    y = lax.bitcast_convert_type(i, jnp.float32)
    for _ in range(3):
        y = y * (jnp.float32(1.5) - jnp.float32(0.5) * x * y * y)
    return y


EC = 256


def _transpose_tables(ent_t, rel_t, ent_tail, rel_tail):
    D, V = ent_t.shape
    W = 2 * D
    NW = NC * NS
    mesh = plsc.VectorSubcoreMesh(
        core_axis_name="c", subcore_axis_name="s", num_cores=NC, num_subcores=NS
    )
    nstd = V // EC
    TT = 30
    cpt = nstd // TT
    e_mid = nstd * EC
    e_hi = e_mid + EC // 2
    nlast = (V - e_hi) // 2
    assert nstd == TT * cpt and nlast == ent_tail.shape[0]

    @functools.partial(
        pl.kernel,
        out_type=(
            jax.ShapeDtypeStruct((V // 2, W), jnp.float32),
            jax.ShapeDtypeStruct((V // 2, W), jnp.float32),
        ),
        mesh=mesh,
        compiler_params=pltpu.CompilerParams(
            needs_layout_passes=False, use_tc_tiling_on_sc=True
        ),
        scratch_types=[
            [pltpu.VMEM((D, 1, EC), jnp.float32)] * 2,
            [pltpu.VMEM((EC // 2, W), jnp.float32)] * 2,
            pltpu.SemaphoreType.DMA,
            pltpu.SemaphoreType.DMA,
        ],
    )
    def tk(ent_hbm, rel_hbm, etail_hbm, rtail_hbm, oent_hbm, orel_hbm,
           stage_v, blk_v, isem, osem):
        wid = lax.axis_index("s") * NC + lax.axis_index("c")
        iota = lax.iota(jnp.int32, L)
        riota = lax.shift_right_logical(iota, 1)
        pcol = (iota & 1) * D

        def transpose_slab(b, ng, d_ref=None):
            def trans_d(d, _):
                cidx = pcol + d
                for g in range(ng):
                    v = stage_v[b][d, 0, pl.ds(g * L, L)]
                    plsc.store_scatter(
                        blk_v[b], [g * (L // 2) + riota, cidx], v
                    )
                return 0
            lax.fori_loop(0, D, trans_d, 0)

        @pl.when(wid < TT)
        def bulk():
            def estart(t):
                c = wid * cpt + (t % cpt)
                return pl.multiple_of(c * EC, EC)

            def src(t):
                return ent_hbm if t < cpt else rel_hbm

            def dst(t):
                return oent_hbm if t < cpt else orel_hbm

            def fire_in(t):
                b = t % 2
                return pltpu.async_copy(
                    src(t).at[:, pl.ds(estart(t), EC)], stage_v[b].at[:, 0],
                    isem,
                )

            inflight = [fire_in(0), fire_in(1)]
            out_pending = []
            for t in range(2 * cpt):
                b = t % 2
                inflight.pop(0).wait()
                if len(out_pending) >= 2:
                    out_pending.pop(0).wait()
                transpose_slab(b, EC // L)
                row0 = pl.multiple_of(
                    lax.shift_right_logical(estart(t), 1), EC // 2
                )
                out_pending.append(
                    pltpu.async_copy(
                        blk_v[b], dst(t).at[pl.ds(row0, EC // 2)], osem
                    )
                )
                if t + 2 < 2 * cpt:
                    inflight.append(fire_in(t + 2))
            for cdesc in out_pending:
                cdesc.wait()

        HC = EC // 2
        for w, thbm, lhbm, ohbm in (
            (TT, ent_hbm, etail_hbm, oent_hbm),
            (TT + 1, rel_hbm, rtail_hbm, orel_hbm),
        ):
            @pl.when(wid == w)
            def tail_work(thbm=thbm, lhbm=lhbm, ohbm=ohbm):
                pltpu.sync_copy(
                    thbm.at[:, pl.ds(e_mid, HC)],
                    stage_v[0].at[:, 0, pl.ds(0, HC)],
                )
                transpose_slab(0, HC // L)
                pltpu.sync_copy(lhbm, blk_v[0].at[pl.ds(HC // 2, nlast)])
                pltpu.sync_copy(
                    blk_v[0].at[pl.ds(0, HC // 2 + nlast)],
                    ohbm.at[pl.ds(e_mid // 2, HC // 2 + nlast)],
                )

    return tk(ent_t, rel_t, ent_tail, rel_tail)


def kernel(triplet_idx, entity_embedding, relation_embedding):
    B = triplet_idx.shape[0]
    D = entity_embedding.shape[1]
    W = 2 * D
    NW = NC * NS
    bpw = B // NW
    nch = bpw // CH
    qv = D // L

    mesh = plsc.VectorSubcoreMesh(
        core_axis_name="c", subcore_axis_name="s", num_cores=NC, num_subcores=NS
    )

    @functools.partial(
        pl.kernel,
        out_type=jax.ShapeDtypeStruct((B,), jnp.float32),
        mesh=mesh,
        compiler_params=pltpu.CompilerParams(
            needs_layout_passes=False, use_tc_tiling_on_sc=True
        ),
        scratch_types=[
            [pltpu.VMEM((bpw,), jnp.int32)] * 3,
            [pltpu.VMEM((nch, 1, CH), jnp.int32)] * 3,
            [pltpu.VMEM((nch, 1, CH), jnp.int32)] * 3,
            [pltpu.VMEM((NBUF, CH, 1, W), jnp.float32)] * 3,
            pltpu.VMEM((bpw,), jnp.float32),
            pltpu.SemaphoreType.DMA,
        ],
    )
    def k(tri_hbm, ent_hbm, rel_hbm, out_hbm, tcol_v, idx_v, off_v, rows_v,
          outv, sem):
        wid = lax.axis_index("s") * NC + lax.axis_index("c")
        base = wid * bpw

        for c in range(3):
            pltpu.sync_copy(tri_hbm.at[pl.ds(c * B + base, bpw)], tcol_v[c])
        iota = lax.iota(jnp.int32, L)
        for g in range(bpw // L):
            j, o = divmod(g * L, CH)
            for c in range(3):
                col = tcol_v[c][pl.ds(g * L, L)]
                idx_v[c][j, 0, pl.ds(o, L)] = lax.shift_right_logical(col, 1)
                off_v[c][j, 0, pl.ds(o, L)] = (col & 1) * D

        def fire(j):
            b = j % NBUF
            return (
                pltpu.async_copy(ent_hbm.at[idx_v[0].at[j, 0]], rows_v[0].at[b, :, 0], sem),
                pltpu.async_copy(rel_hbm.at[idx_v[1].at[j, 0]], rows_v[1].at[b, :, 0], sem),
                pltpu.async_copy(ent_hbm.at[idx_v[2].at[j, 0]], rows_v[2].at[b, :, 0], sem),
            )

        eps = jnp.float32(1e-6)
        tiny = jnp.float32(1e-24)
        inflight = [fire(j) for j in range(min(NBUF, nch))]

        for j in range(nch):
            b = j % NBUF
            for cdesc in inflight[0]:
                cdesc.wait()
            inflight.pop(0)

            def group(g, _, j=j, b=b):
                hov = off_v[0][j, 0, pl.ds(g * L, L)]
                rov = off_v[1][j, 0, pl.ds(g * L, L)]
                tov = off_v[2][j, 0, pl.ds(g * L, L)]
                hsv = jnp.zeros((L,), jnp.float32)
                tsv = jnp.zeros((L,), jnp.float32)
                for k in range(L):
                    i = g * L + k
                    ho = hov[k]
                    to = tov[k]
                    h = [rows_v[0][b, i, 0, pl.ds(ho + q * L, L)] for q in range(qv)]
                    t = [rows_v[2][b, i, 0, pl.ds(to + q * L, L)] for q in range(qv)]
                    hh = h[0] * h[0]
                    tt = t[0] * t[0]
                    for q in range(1, qv):
                        hh = hh + h[q] * h[q]
                        tt = tt + t[q] * t[q]
                    hsv = jnp.where(iota == k, jnp.sum(hh), hsv)
                    tsv = jnp.where(iota == k, jnp.sum(tt), tsv)
                ihv = _rsqrt(jnp.maximum(hsv, tiny))
                itv = _rsqrt(jnp.maximum(tsv, tiny))
                acc = jnp.zeros((L,), jnp.float32)
                for k in range(L):
                    i = g * L + k
                    ih = ihv[k]
                    it = itv[k]
                    ho = hov[k]
                    ro = rov[k]
                    to = tov[k]
                    h = [rows_v[0][b, i, 0, pl.ds(ho + q * L, L)] for q in range(qv)]
                    t = [rows_v[2][b, i, 0, pl.ds(to + q * L, L)] for q in range(qv)]
                    r = [rows_v[1][b, i, 0, pl.ds(ro + q * L, L)] for q in range(qv)]
                    s = jnp.abs(h[0] * ih + (r[0] + eps) - t[0] * it)
                    for q in range(1, qv):
                        s = s + jnp.abs(h[q] * ih + (r[q] + eps) - t[q] * it)
                    acc = jnp.where(iota == k, jnp.sum(s), acc)
                outv[pl.ds(j * CH + g * L, L)] = acc
                return 0
            lax.fori_loop(0, CH // L, group, 0)

            if j + NBUF < nch:
                inflight.append(fire(j + NBUF))

        pltpu.sync_copy(outv, out_hbm.at[pl.ds(base, bpw)])

    V = entity_embedding.shape[0]
    e_hi = (V // EC) * EC + EC // 2
    ent2, rel2 = _transpose_tables(
        entity_embedding.T,
        relation_embedding.T,
        entity_embedding[e_hi:].reshape(-1, W),
        relation_embedding[e_hi:].reshape(-1, W),
    )
    tri_flat = triplet_idx.T.reshape(-1)
    return k(tri_flat, ent2, rel2)


# speedup vs baseline: 2.6135x; 2.6135x over previous
import functools

import jax
import jax.numpy as jnp
from jax import lax
from jax.experimental import pallas as pl
from jax.experimental.pallas import tpu as pltpu
from jax.experimental.pallas import tpu_sc as plsc

NC = 2
NS = 16
L = 16
CH = 128
NBUF = 2


def _perm(v, idx):
    dnums = lax.GatherDimensionNumbers(
        offset_dims=(), collapsed_slice_dims=(0,), start_index_map=(0,)
    )
    return lax.gather(
        v, idx[:, None], dnums, (1,),
        mode=lax.GatherScatterMode.PROMISE_IN_BOUNDS,
    )


def _rsqrt(x):
    i = lax.bitcast_convert_type(x, jnp.int32)
    i = jnp.int32(0x5F3759DF) - lax.shift_right_logical(i, 1)
    y = lax.bitcast_convert_type(i, jnp.float32)
    for _ in range(3):
        y = y * (jnp.float32(1.5) - jnp.float32(0.5) * x * y * y)
    return y


EC = 256


def _transpose_tables(ent_t, rel_t, ent_tail, rel_tail):
    D, V = ent_t.shape
    W = 2 * D
    NW = NC * NS
    mesh = plsc.VectorSubcoreMesh(
        core_axis_name="c", subcore_axis_name="s", num_cores=NC, num_subcores=NS
    )
    nstd = V // EC
    TT = 30
    cpt = nstd // TT
    e_mid = nstd * EC
    e_hi = e_mid + EC // 2
    nlast = (V - e_hi) // 2
    assert nstd == TT * cpt and nlast == ent_tail.shape[0]

    @functools.partial(
        pl.kernel,
        out_type=(
            jax.ShapeDtypeStruct((V // 2, W), jnp.float32),
            jax.ShapeDtypeStruct((V // 2, W), jnp.float32),
        ),
        mesh=mesh,
        compiler_params=pltpu.CompilerParams(
            needs_layout_passes=False, use_tc_tiling_on_sc=True
        ),
        scratch_types=[
            [pltpu.VMEM((D, 1, EC), jnp.float32)] * 2,
            [pltpu.VMEM((EC // 2, 1, W), jnp.float32)] * 2,
            pltpu.SemaphoreType.DMA,
            pltpu.SemaphoreType.DMA,
        ],
    )
    def tk(ent_hbm, rel_hbm, etail_hbm, rtail_hbm, oent_hbm, orel_hbm,
           stage_v, blk_v, isem, osem):
        wid = lax.axis_index("s") * NC + lax.axis_index("c")
        iota = lax.iota(jnp.int32, L)
        riota = lax.shift_right_logical(iota, 1)
        pcol = (iota & 1) * D

        def transpose_slab(b, ng):
            def block(m, _, b=b, ng=ng):
                tb = m // ng
                eg = m - tb * ng
                d0 = tb * L
                e0 = eg * L
                vs = [stage_v[b][d0 + i, 0, pl.ds(e0, L)] for i in range(L)]
                for s in (1, 2, 4, 8):
                    pidx = iota ^ s
                    vs = [
                        jnp.where(
                            (iota & s) == (i & s),
                            vs[i],
                            _perm(vs[i ^ s], pidx),
                        )
                        for i in range(L)
                    ]
                row0 = eg * (L // 2)
                for j in range(L):
                    blk_v[b][row0 + j // 2, 0, pl.ds((j & 1) * D + d0, L)] = vs[j]
                return 0
            lax.fori_loop(0, (D // L) * ng, block, 0)

        @pl.when(wid < TT)
        def bulk():
            def estart(t):
                c = wid * cpt + (t % cpt)
                return pl.multiple_of(c * EC, EC)

            def src(t):
                return ent_hbm if t < cpt else rel_hbm

            def dst(t):
                return oent_hbm if t < cpt else orel_hbm

            def fire_in(t):
                b = t % 2
                return pltpu.async_copy(
                    src(t).at[:, pl.ds(estart(t), EC)], stage_v[b].at[:, 0],
                    isem,
                )

            inflight = [fire_in(0), fire_in(1)]
            out_pending = []
            for t in range(2 * cpt):
                b = t % 2
                inflight.pop(0).wait()
                if len(out_pending) >= 2:
                    out_pending.pop(0).wait()
                transpose_slab(b, EC // L)
                row0 = pl.multiple_of(
                    lax.shift_right_logical(estart(t), 1), EC // 2
                )
                out_pending.append(
                    pltpu.async_copy(
                        blk_v[b].at[:, 0], dst(t).at[pl.ds(row0, EC // 2)], osem
                    )
                )
                if t + 2 < 2 * cpt:
                    inflight.append(fire_in(t + 2))
            for cdesc in out_pending:
                cdesc.wait()

        HC = EC // 2
        for w, thbm, lhbm, ohbm in (
            (TT, ent_hbm, etail_hbm, oent_hbm),
            (TT + 1, rel_hbm, rtail_hbm, orel_hbm),
        ):
            @pl.when(wid == w)
            def tail_work(thbm=thbm, lhbm=lhbm, ohbm=ohbm):
                pltpu.sync_copy(
                    thbm.at[:, pl.ds(e_mid, HC)],
                    stage_v[0].at[:, 0, pl.ds(0, HC)],
                )
                transpose_slab(0, HC // L)
                pltpu.sync_copy(lhbm, blk_v[0].at[pl.ds(HC // 2, nlast), 0])
                pltpu.sync_copy(
                    blk_v[0].at[pl.ds(0, HC // 2 + nlast), 0],
                    ohbm.at[pl.ds(e_mid // 2, HC // 2 + nlast)],
                )

    return tk(ent_t, rel_t, ent_tail, rel_tail)


def kernel(triplet_idx, entity_embedding, relation_embedding):
    B = triplet_idx.shape[0]
    D = entity_embedding.shape[1]
    W = 2 * D
    NW = NC * NS
    bpw = B // NW
    nch = bpw // CH
    qv = D // L

    mesh = plsc.VectorSubcoreMesh(
        core_axis_name="c", subcore_axis_name="s", num_cores=NC, num_subcores=NS
    )

    @functools.partial(
        pl.kernel,
        out_type=jax.ShapeDtypeStruct((B,), jnp.float32),
        mesh=mesh,
        compiler_params=pltpu.CompilerParams(
            needs_layout_passes=False, use_tc_tiling_on_sc=True
        ),
        scratch_types=[
            [pltpu.VMEM((bpw,), jnp.int32)] * 3,
            [pltpu.VMEM((nch, 1, CH), jnp.int32)] * 3,
            [pltpu.VMEM((nch, 1, CH), jnp.int32)] * 3,
            [pltpu.VMEM((NBUF, CH, 1, W), jnp.float32)] * 3,
            pltpu.VMEM((bpw,), jnp.float32),
            pltpu.SemaphoreType.DMA,
        ],
    )
    def k(tri_hbm, ent_hbm, rel_hbm, out_hbm, tcol_v, idx_v, off_v, rows_v,
          outv, sem):
        wid = lax.axis_index("s") * NC + lax.axis_index("c")
        base = wid * bpw

        for c in range(3):
            pltpu.sync_copy(tri_hbm.at[pl.ds(c * B + base, bpw)], tcol_v[c])
        iota = lax.iota(jnp.int32, L)
        for g in range(bpw // L):
            j, o = divmod(g * L, CH)
            for c in range(3):
                col = tcol_v[c][pl.ds(g * L, L)]
                idx_v[c][j, 0, pl.ds(o, L)] = lax.shift_right_logical(col, 1)
                off_v[c][j, 0, pl.ds(o, L)] = (col & 1) * D

        def fire(j):
            b = j % NBUF
            return (
                pltpu.async_copy(ent_hbm.at[idx_v[0].at[j, 0]], rows_v[0].at[b, :, 0], sem),
                pltpu.async_copy(rel_hbm.at[idx_v[1].at[j, 0]], rows_v[1].at[b, :, 0], sem),
                pltpu.async_copy(ent_hbm.at[idx_v[2].at[j, 0]], rows_v[2].at[b, :, 0], sem),
            )

        eps = jnp.float32(1e-6)
        tiny = jnp.float32(1e-24)
        inflight = [fire(j) for j in range(min(NBUF, nch))]

        for j in range(nch):
            b = j % NBUF
            for cdesc in inflight[0]:
                cdesc.wait()
            inflight.pop(0)

            def group(g, _, j=j, b=b):
                hov = off_v[0][j, 0, pl.ds(g * L, L)]
                rov = off_v[1][j, 0, pl.ds(g * L, L)]
                tov = off_v[2][j, 0, pl.ds(g * L, L)]
                hsv = jnp.zeros((L,), jnp.float32)
                tsv = jnp.zeros((L,), jnp.float32)
                for k in range(L):
                    i = g * L + k
                    ho = hov[k]
                    to = tov[k]
                    h = [rows_v[0][b, i, 0, pl.ds(ho + q * L, L)] for q in range(qv)]
                    t = [rows_v[2][b, i, 0, pl.ds(to + q * L, L)] for q in range(qv)]
                    hh = h[0] * h[0]
                    tt = t[0] * t[0]
                    for q in range(1, qv):
                        hh = hh + h[q] * h[q]
                        tt = tt + t[q] * t[q]
                    hsv = jnp.where(iota == k, jnp.sum(hh), hsv)
                    tsv = jnp.where(iota == k, jnp.sum(tt), tsv)
                ihv = _rsqrt(jnp.maximum(hsv, tiny))
                itv = _rsqrt(jnp.maximum(tsv, tiny))
                acc = jnp.zeros((L,), jnp.float32)
                for k in range(L):
                    i = g * L + k
                    ih = ihv[k]
                    it = itv[k]
                    ho = hov[k]
                    ro = rov[k]
                    to = tov[k]
                    h = [rows_v[0][b, i, 0, pl.ds(ho + q * L, L)] for q in range(qv)]
                    t = [rows_v[2][b, i, 0, pl.ds(to + q * L, L)] for q in range(qv)]
                    r = [rows_v[1][b, i, 0, pl.ds(ro + q * L, L)] for q in range(qv)]
                    s = jnp.abs(h[0] * ih + (r[0] + eps) - t[0] * it)
                    for q in range(1, qv):
                        s = s + jnp.abs(h[q] * ih + (r[q] + eps) - t[q] * it)
                    acc = jnp.where(iota == k, jnp.sum(s), acc)
                outv[pl.ds(j * CH + g * L, L)] = acc
                return 0
            lax.fori_loop(0, CH // L, group, 0)

            if j + NBUF < nch:
                inflight.append(fire(j + NBUF))

        pltpu.sync_copy(outv, out_hbm.at[pl.ds(base, bpw)])

    V = entity_embedding.shape[0]
    e_hi = (V // EC) * EC + EC // 2
    ent2, rel2 = _transpose_tables(
        entity_embedding.T,
        relation_embedding.T,
        entity_embedding[e_hi:].reshape(-1, W),
        relation_embedding[e_hi:].reshape(-1, W),
    )
    tri_flat = triplet_idx.T.reshape(-1)
    return k(tri_flat, ent2, rel2)
